# Initial kernel scaffold; baseline (speedup 1.0000x reference)
import jax
import jax.numpy as jnp
from jax.experimental import pallas as pl


def kernel(node, edge_index, rbf, rsh, W_src, W_dst, W_rbf, W_alpha, alpha_dot, W_sepval, w_dtp2, W_vlin, W_proj):
    raise NotImplementedError("write your pallas kernel here")


# speedup vs baseline: 32.6416x; 32.6416x over previous
import dataclasses
import functools

import jax
import jax.numpy as jnp
from jax import lax
from jax.experimental import pallas as pl
from jax.experimental.pallas import tpu as pltpu
from jax.experimental.pallas import tpu_sc as plsc

N = 10000
E = 160000
D = 128
DE = 16
H = 4
A = 32

NC = 2
NS = 16
NW = NC * NS

BN = 2000
BNF = 2048
BE = 4000
CH = 128
NCHUNK = E // CH
GK = 2
GGROUPS = NCHUNK // GK
GITER = (GGROUPS + NW - 1) // NW
SK = 2
SGROUPS = NCHUNK // SK
SITER = (SGROUPS + NW - 1) // NW
ROWS_N = (N // NS) // 8 * 8
ROWS_TAIL = N - NS * ROWS_N
CW = D + 16

_F32 = jnp.float32


def _node_body(node_ref, wsrcT_ref, wdstT_ref, msrc_ref, mdst_ref):
    x = node_ref[...]
    mu = jnp.mean(x, axis=-1, keepdims=True)
    xc = x - mu
    var = jnp.mean(xc * xc, axis=-1, keepdims=True)
    xn = xc * lax.rsqrt(var + 1e-5)
    msrc_ref[...] = jnp.dot(xn, wsrcT_ref[...], preferred_element_type=_F32)
    mdst_ref[...] = jnp.dot(xn, wdstT_ref[...], preferred_element_type=_F32)


def _node_pass(node, wsrcT, wdstT):
    return pl.pallas_call(
        _node_body,
        grid=(N // BN,),
        in_specs=[
            pl.BlockSpec((BN, D), lambda i: (i, 0)),
            pl.BlockSpec((D, D), lambda i: (0, 0)),
            pl.BlockSpec((D, D), lambda i: (0, 0)),
        ],
        out_specs=[
            pl.BlockSpec((BN, D), lambda i: (i, 0)),
            pl.BlockSpec((BN, D), lambda i: (i, 0)),
        ],
        out_shape=[jax.ShapeDtypeStruct((N, D), _F32)] * 2,
    )(node, wsrcT, wdstT)


def _sc_gather_body(msrc_hbm, mdst_hbm, srcix_hbm, dstix_hbm, g1_hbm, g2_hbm,
                    *rest):
    idxa_v = rest[0:GK]
    idxb_v = rest[GK:2 * GK]
    bufa_v, bufb_v, sem = rest[2 * GK:]
    c = lax.axis_index("c")
    s = lax.axis_index("s")
    wid = s * NC + c

    @pl.loop(0, GITER)
    def _(i):
        gid = wid + i * NW

        @pl.when(gid < GGROUPS)
        def _():
            for j in range(GK):
                pltpu.sync_copy(
                    srcix_hbm.at[pl.ds(gid * (GK * CH) + j * CH, CH)],
                    idxa_v[j])
                pltpu.sync_copy(
                    dstix_hbm.at[pl.ds(gid * (GK * CH) + j * CH, CH)],
                    idxb_v[j])
            cps = []
            for j in range(GK):
                cps.append(pltpu.async_copy(
                    msrc_hbm.at[idxa_v[j]],
                    bufa_v.at[pl.ds(j * CH, CH)], sem))
                cps.append(pltpu.async_copy(
                    mdst_hbm.at[idxb_v[j]],
                    bufb_v.at[pl.ds(j * CH, CH)], sem))
            for cp in cps:
                cp.wait()
            base = gid * (GK * CH)
            w1 = pltpu.async_copy(bufa_v, g1_hbm.at[pl.ds(base, GK * CH)], sem)
            w2 = pltpu.async_copy(bufb_v, g2_hbm.at[pl.ds(base, GK * CH)], sem)
            w1.wait()
            w2.wait()


def _sc_gather(msrc, mdst, src3d, dst3d):
    mesh = plsc.VectorSubcoreMesh(core_axis_name="c", subcore_axis_name="s")
    f = pl.kernel(
        _sc_gather_body,
        out_type=(jax.ShapeDtypeStruct((E, D), _F32),
                  jax.ShapeDtypeStruct((E, D), _F32)),
        mesh=mesh,
        scratch_types=(
            [pltpu.VMEM((CH,), jnp.int32) for _ in range(2 * GK)] + [
                pltpu.VMEM((GK * CH, D), _F32),
                pltpu.VMEM((GK * CH, D), _F32),
                pltpu.SemaphoreType.DMA,
            ]
        ),
    )
    return f(msrc, mdst, src3d, dst3d)


def _edge_body(g1_ref, g2_ref, rbf_ref, rsh_ref, wrbfT_ref, walphaT_ref,
               dvec_ref, wsepT_ref, wdtp2_ref, wvlinT_ref, e4_ref, e4pad_ref,
               attn_ref, wexp_ref):
    msg = g1_ref[...] + g2_ref[...]
    rbf = rbf_ref[...]
    rsh = rsh_ref[...]
    w1 = jnp.dot(rbf, wrbfT_ref[...], preferred_element_type=_F32)
    s1 = jnp.sum(w1 * rsh, axis=-1, keepdims=True)
    m = msg * s1
    lr = jnp.dot(m, walphaT_ref[...], preferred_element_type=_F32)
    lr = jnp.where(lr >= 0, lr, 0.2 * lr)
    wexp = jnp.exp(jnp.dot(lr, dvec_ref[...], preferred_element_type=_F32))
    v = jnp.dot(m, wsepT_ref[...], preferred_element_type=_F32)
    v = v * jax.nn.sigmoid(v)
    s2 = jnp.dot(rsh, wdtp2_ref[...], preferred_element_type=_F32)
    v = v * s2
    v = jnp.dot(v, wvlinT_ref[...], preferred_element_type=_F32)
    attn_ref[...] = v * jnp.dot(wexp, e4_ref[...], preferred_element_type=_F32)
    wexp_ref[...] = jnp.dot(wexp, e4pad_ref[...], preferred_element_type=_F32)


def _edge_pass(g1, g2, rbf, rsh, wrbfT, walphaT, dvec, wsepT, wdtp2c, wvlinT,
               e4, e4pad):
    full = lambda shape: pl.BlockSpec(shape, lambda i: tuple(0 for _ in shape))
    return pl.pallas_call(
        _edge_body,
        grid=(E // BE,),
        in_specs=[
            pl.BlockSpec((BE, D), lambda i: (i, 0)),
            pl.BlockSpec((BE, D), lambda i: (i, 0)),
            pl.BlockSpec((BE, DE), lambda i: (i, 0)),
            pl.BlockSpec((BE, DE), lambda i: (i, 0)),
            full((DE, DE)),
            full((D, D)),
            full((D, H)),
            full((D, D)),
            full((DE, 1)),
            full((D, D)),
            full((H, D)),
            full((H, DE)),
        ],
        out_specs=[
            pl.BlockSpec((BE, D), lambda i: (i, 0)),
            pl.BlockSpec((BE, DE), lambda i: (i, 0)),
        ],
        out_shape=[jax.ShapeDtypeStruct((E, D), _F32),
                   jax.ShapeDtypeStruct((E, DE), _F32)],
    )(g1, g2, rbf, rsh, wrbfT, walphaT, dvec, wsepT, wdtp2c, wvlinT, e4, e4pad)


def _sc_scatter_body(attn_hbm, dstix_hbm, zeros_hbm, out_hbm, *rest):
    idx_v = rest[0:SK]
    rows_v, table_sh, sem = rest[SK:]
    c = lax.axis_index("c")
    s = lax.axis_index("s")
    wid = s * NC + c

    pltpu.sync_copy(zeros_hbm.at[pl.ds(s * ROWS_N, ROWS_N)],
                    table_sh.at[pl.ds(s * ROWS_N, ROWS_N)])

    @pl.when(s == 0)
    def _():
        pltpu.sync_copy(zeros_hbm.at[pl.ds(NS * ROWS_N, ROWS_TAIL)],
                        table_sh.at[pl.ds(NS * ROWS_N, ROWS_TAIL)])

    plsc.subcore_barrier()

    @pl.loop(0, SITER)
    def _(i):
        gid = wid + i * NW

        @pl.when(gid < SGROUPS)
        def _():
            for j in range(SK):
                pltpu.sync_copy(
                    dstix_hbm.at[pl.ds(gid * (SK * CH) + j * CH, CH)],
                    idx_v[j])
            cp = pltpu.async_copy(
                attn_hbm.at[pl.ds(gid * (SK * CH), SK * CH)], rows_v, sem)
            cp.wait()
            for j in range(SK):
                pltpu.sync_copy(rows_v.at[pl.ds(j * CH, CH)],
                                table_sh.at[idx_v[j]], add=True)

    plsc.subcore_barrier()
    pltpu.sync_copy(table_sh.at[pl.ds(s * ROWS_N, ROWS_N)],
                    out_hbm.at[c, pl.ds(s * ROWS_N, ROWS_N)])

    @pl.when(s == 0)
    def _():
        pltpu.sync_copy(table_sh.at[pl.ds(NS * ROWS_N, ROWS_TAIL)],
                        out_hbm.at[c, pl.ds(NS * ROWS_N, ROWS_TAIL)])


def _sc_scatter(attn, dst1d, zeros_tab):
    mesh = plsc.VectorSubcoreMesh(core_axis_name="c", subcore_axis_name="s")
    f = pl.kernel(
        _sc_scatter_body,
        out_type=jax.ShapeDtypeStruct((NC, N, D), _F32),
        mesh=mesh,
        scratch_types=(
            [pltpu.VMEM((CH,), jnp.int32) for _ in range(SK)] + [
                pltpu.VMEM((SK * CH, D), _F32),
                pltpu.VMEM_SHARED((N, D), _F32),
                pltpu.SemaphoreType.DMA,
            ]
        ),
    )
    return f(attn, dst1d, zeros_tab)


WK = 5
WGROUPS = NCHUNK // WK
WITER = (WGROUPS + NW - 1) // NW
NP = N * H
NPP = 40960


def _sc_wsum_body(wexp_hbm, dstix_hbm, wout_hbm, *rest):
    idx_v = rest[0:WK]
    wrows_v, wtab_v, sem = rest[WK:]
    c = lax.axis_index("c")
    s = lax.axis_index("s")
    wid = s * NC + c

    zvec = jnp.zeros((16,), _F32)

    @pl.loop(0, NPP // 16)
    def _(i):
        wtab_v[pl.ds(i * 16, 16)] = zvec

    lane = lax.iota(jnp.int32, 16)

    @pl.loop(0, WITER)
    def _(i):
        gid = wid + i * NW

        @pl.when(gid < WGROUPS)
        def _():
            for j in range(WK):
                pltpu.sync_copy(
                    dstix_hbm.at[pl.ds(gid * (WK * CH) + j * CH, CH)],
                    idx_v[j])
            cpw = pltpu.async_copy(
                wexp_hbm.at[pl.ds(gid * (WK * CH * DE), WK * CH * DE)],
                wrows_v, sem)
            cpw.wait()
            for j in range(WK):
                @pl.loop(0, CH)
                def _(e):
                    x = wrows_v[pl.ds(j * (CH * DE) + e * DE, 16)]
                    dstb = plsc.load_gather(
                        idx_v[j], [jnp.full((16,), 0, jnp.int32) + e])
                    plsc.addupdate_scatter(wtab_v, [dstb * H + lane], x)

    pltpu.sync_copy(wtab_v, wout_hbm.at[pl.ds(wid * NPP, NPP)])


def _sc_wsum(wexp1d, dst1d):
    mesh = plsc.VectorSubcoreMesh(core_axis_name="c", subcore_axis_name="s")
    cp = pltpu.CompilerParams()
    if "needs_layout_passes" in pltpu.CompilerParams.__dataclass_fields__:
        cp = dataclasses.replace(cp, needs_layout_passes=False)
    f = pl.kernel(
        _sc_wsum_body,
        compiler_params=cp,
        out_type=jax.ShapeDtypeStruct((NW * NPP,), _F32),
        mesh=mesh,
        scratch_types=(
            [pltpu.VMEM((CH,), jnp.int32) for _ in range(WK)] + [
                pltpu.VMEM((WK * CH * DE,), _F32),
                pltpu.VMEM((NPP,), _F32),
                pltpu.SemaphoreType.DMA,
            ]
        ),
    )
    return f(wexp1d, dst1d)


def _final_body(acc0_ref, acc1_ref, wparts_ref, node_ref, wprojT_ref,
                brep_ref, lt_ref, rsel_ref, out_ref):
    ssum = acc0_ref[...] + acc1_ref[...]
    w2 = jnp.sum(wparts_ref[...], axis=0)
    inv2 = 1.0 / (w2 + 1e-16)
    rep = jnp.dot(brep_ref[...], inv2, preferred_element_type=_F32)
    inv128 = jnp.dot(rep * lt_ref[...], rsel_ref[...],
                     preferred_element_type=_F32)
    out_ref[...] = node_ref[...] + jnp.dot(
        ssum * inv128, wprojT_ref[...], preferred_element_type=_F32)


def _final_pass(acc0, acc1, wparts2d, node, wprojT, brep, lt, rsel):
    full = lambda shape: pl.BlockSpec(shape, lambda i: tuple(0 for _ in shape))
    wrows = (BNF * H) // 128
    return pl.pallas_call(
        _final_body,
        grid=(pl.cdiv(N, BNF),),
        in_specs=[
            pl.BlockSpec((BNF, D), lambda i: (i, 0)),
            pl.BlockSpec((BNF, D), lambda i: (i, 0)),
            pl.BlockSpec((NW, wrows, 128), lambda i: (0, i, 0)),
            pl.BlockSpec((BNF, D), lambda i: (i, 0)),
            full((D, D)),
            full((BNF, wrows)),
            full((BNF, D)),
            full((D, D)),
        ],
        out_specs=pl.BlockSpec((BNF, D), lambda i: (i, 0)),
        out_shape=jax.ShapeDtypeStruct((N, D), _F32),
    )(acc0, acc1, wparts2d, node, wprojT, brep, lt, rsel)


@jax.jit
def kernel(node, edge_index, rbf, rsh, W_src, W_dst, W_rbf, W_alpha,
           alpha_dot, W_sepval, w_dtp2, W_vlin, W_proj):
    src1d = edge_index[0]
    dst1d = edge_index[1]
    eye4 = jnp.eye(H, dtype=_F32)
    e4 = jnp.repeat(eye4, A, axis=1)
    e4pad = jnp.concatenate(
        [eye4, jnp.zeros((H, DE - H), _F32)], axis=1)
    dvec = (alpha_dot[0][:, :, None] * eye4[:, None, :]).reshape(H * A, H)
    zeros_tab = jnp.zeros((N, D), _F32)

    msrc, mdst = _node_pass(node, W_src.T, W_dst.T)
    g1, g2 = _sc_gather(msrc, mdst, src1d, dst1d)
    attn, wexp = _edge_pass(g1, g2, rbf, rsh, W_rbf.T, W_alpha.T, dvec,
                            W_sepval.T, w_dtp2.reshape(DE, 1), W_vlin.T, e4,
                            e4pad)
    acc = _sc_scatter(attn, dst1d, zeros_tab)
    wparts = _sc_wsum(wexp.reshape(E * DE), dst1d)
    wrows = (BNF * H) // 128
    brep = jnp.repeat(jnp.eye(wrows, dtype=_F32), 128 // H, axis=0)
    lt = jnp.tile(jnp.repeat(jnp.eye(128 // H, dtype=_F32), H, axis=1),
                  (BNF // (128 // H), 1))
    rsel = jnp.tile(jnp.repeat(eye4, A, axis=1), (A, 1))
    return _final_pass(acc[0], acc[1], wparts.reshape(NW, NPP // 128, 128),
                       node, W_proj.T, brep, lt, rsel)


# speedup vs baseline: 33.1313x; 1.0150x over previous
import dataclasses
import functools

import jax
import jax.numpy as jnp
from jax import lax
from jax.experimental import pallas as pl
from jax.experimental.pallas import tpu as pltpu
from jax.experimental.pallas import tpu_sc as plsc

N = 10000
E = 160000
D = 128
DE = 16
H = 4
A = 32

NC = 2
NS = 16
NW = NC * NS

BN = 2000
BNF = 2048
BE = 4000
CH = 128
NCHUNK = E // CH
GK = 2
GGROUPS = NCHUNK // GK
GITER = (GGROUPS + NW - 1) // NW
SK = 2
SGROUPS = NCHUNK // SK
SITER = (SGROUPS + NW - 1) // NW
ROWS_N = (N // NS) // 8 * 8
ROWS_TAIL = N - NS * ROWS_N
CW = D + 16

_F32 = jnp.float32


def _node_body(node_ref, wsrcT_ref, wdstT_ref, msrc_ref, mdst_ref):
    x = node_ref[...]
    mu = jnp.mean(x, axis=-1, keepdims=True)
    xc = x - mu
    var = jnp.mean(xc * xc, axis=-1, keepdims=True)
    xn = xc * lax.rsqrt(var + 1e-5)
    msrc_ref[...] = jnp.dot(xn, wsrcT_ref[...], preferred_element_type=_F32)
    mdst_ref[...] = jnp.dot(xn, wdstT_ref[...], preferred_element_type=_F32)


def _node_pass(node, wsrcT, wdstT):
    return pl.pallas_call(
        _node_body,
        grid=(N // BN,),
        in_specs=[
            pl.BlockSpec((BN, D), lambda i: (i, 0)),
            pl.BlockSpec((D, D), lambda i: (0, 0)),
            pl.BlockSpec((D, D), lambda i: (0, 0)),
        ],
        out_specs=[
            pl.BlockSpec((BN, D), lambda i: (i, 0)),
            pl.BlockSpec((BN, D), lambda i: (i, 0)),
        ],
        out_shape=[jax.ShapeDtypeStruct((N, D), _F32)] * 2,
    )(node, wsrcT, wdstT)


CPW = (NCHUNK + NW - 1) // NW


def _sc_gather_body(msrc_hbm, mdst_hbm, srcix_hbm, dstix_hbm, g1_hbm, g2_hbm,
                    idxa_v, idxb_v, bufa0, bufa1, bufb0, bufb1,
                    semg0, semg1, semw0, semw1):
    c = lax.axis_index("c")
    s = lax.axis_index("s")
    wid = s * NC + c
    start = jnp.minimum(wid * CPW, NCHUNK - CPW)
    bufa = (bufa0, bufa1)
    bufb = (bufb0, bufb1)
    semg = (semg0, semg1)
    semw = (semw0, semw1)

    pltpu.sync_copy(srcix_hbm.at[pl.ds(start * CH, CPW * CH)], idxa_v)
    pltpu.sync_copy(dstix_hbm.at[pl.ds(start * CH, CPW * CH)], idxb_v)

    gdesc, wdesc = {}, {}

    def fire_gather(i):
        p = i & 1
        a = pltpu.async_copy(msrc_hbm.at[idxa_v.at[pl.ds(i * CH, CH)]],
                             bufa[p], semg[p])
        b = pltpu.async_copy(mdst_hbm.at[idxb_v.at[pl.ds(i * CH, CH)]],
                             bufb[p], semg[p])
        gdesc[i] = (a, b)

    def fire_write(i):
        p = i & 1
        base = (start + i) * CH
        w1 = pltpu.async_copy(bufa[p], g1_hbm.at[pl.ds(base, CH)], semw[p])
        w2 = pltpu.async_copy(bufb[p], g2_hbm.at[pl.ds(base, CH)], semw[p])
        wdesc[i] = (w1, w2)

    fire_gather(0)
    for i in range(CPW):
        if i >= 1:
            for dsc in wdesc.pop(i - 1):
                dsc.wait()
        if i + 1 < CPW:
            fire_gather(i + 1)
        for dsc in gdesc.pop(i):
            dsc.wait()
        fire_write(i)
    for dsc in wdesc.pop(CPW - 1):
        dsc.wait()


def _sc_gather(msrcp, mdstp, src1d, dst1d):
    mesh = plsc.VectorSubcoreMesh(core_axis_name="c", subcore_axis_name="s")
    f = pl.kernel(
        _sc_gather_body,
        out_type=(jax.ShapeDtypeStruct((E, D), _F32),
                  jax.ShapeDtypeStruct((E, D), _F32)),
        mesh=mesh,
        scratch_types=[
            pltpu.VMEM((CPW * CH,), jnp.int32),
            pltpu.VMEM((CPW * CH,), jnp.int32),
            pltpu.VMEM((CH, D), _F32),
            pltpu.VMEM((CH, D), _F32),
            pltpu.VMEM((CH, D), _F32),
            pltpu.VMEM((CH, D), _F32),
            pltpu.SemaphoreType.DMA,
            pltpu.SemaphoreType.DMA,
            pltpu.SemaphoreType.DMA,
            pltpu.SemaphoreType.DMA,
        ],
    )
    return f(msrcp, mdstp, src1d, dst1d)


def _edge_body(g1_ref, g2_ref, rbf_ref, rsh_ref, wrbfT_ref, walphaT_ref,
               dvec_ref, wsepT_ref, wdtp2_ref, wvlinT_ref, e4_ref, e4pad_ref,
               attn_ref, wexp_ref):
    msg = g1_ref[...] + g2_ref[...]
    rbf = rbf_ref[...]
    rsh = rsh_ref[...]
    w1 = jnp.dot(rbf, wrbfT_ref[...], preferred_element_type=_F32)
    s1 = jnp.sum(w1 * rsh, axis=-1, keepdims=True)
    m = msg * s1
    lr = jnp.dot(m, walphaT_ref[...], preferred_element_type=_F32)
    lr = jnp.where(lr >= 0, lr, 0.2 * lr)
    wexp = jnp.exp(jnp.dot(lr, dvec_ref[...], preferred_element_type=_F32))
    v = jnp.dot(m, wsepT_ref[...], preferred_element_type=_F32)
    v = v * jax.nn.sigmoid(v)
    s2 = jnp.dot(rsh, wdtp2_ref[...], preferred_element_type=_F32)
    v = v * s2
    v = jnp.dot(v, wvlinT_ref[...], preferred_element_type=_F32)
    attn_ref[...] = v * jnp.dot(wexp, e4_ref[...], preferred_element_type=_F32)
    wexp_ref[...] = jnp.dot(wexp, e4pad_ref[...], preferred_element_type=_F32)


def _edge_pass(g1, g2, rbf, rsh, wrbfT, walphaT, dvec, wsepT, wdtp2c, wvlinT,
               e4, e4pad):
    full = lambda shape: pl.BlockSpec(shape, lambda i: tuple(0 for _ in shape))
    return pl.pallas_call(
        _edge_body,
        grid=(E // BE,),
        in_specs=[
            pl.BlockSpec((BE, D), lambda i: (i, 0)),
            pl.BlockSpec((BE, D), lambda i: (i, 0)),
            pl.BlockSpec((BE, DE), lambda i: (i, 0)),
            pl.BlockSpec((BE, DE), lambda i: (i, 0)),
            full((DE, DE)),
            full((D, D)),
            full((D, H)),
            full((D, D)),
            full((DE, 1)),
            full((D, D)),
            full((H, D)),
            full((H, DE)),
        ],
        out_specs=[
            pl.BlockSpec((BE, D), lambda i: (i, 0)),
            pl.BlockSpec((BE, DE), lambda i: (i, 0)),
        ],
        out_shape=[jax.ShapeDtypeStruct((E, D), _F32),
                   jax.ShapeDtypeStruct((E, DE), _F32)],
    )(g1, g2, rbf, rsh, wrbfT, walphaT, dvec, wsepT, wdtp2c, wvlinT, e4, e4pad)


def _sc_scatter_body(attn_hbm, dstix_hbm, zeros_hbm, out_hbm, *rest):
    idx_v = rest[0:SK]
    rows_v, table_sh, sem = rest[SK:]
    c = lax.axis_index("c")
    s = lax.axis_index("s")
    wid = s * NC + c

    pltpu.sync_copy(zeros_hbm.at[pl.ds(s * ROWS_N, ROWS_N)],
                    table_sh.at[pl.ds(s * ROWS_N, ROWS_N)])

    @pl.when(s == 0)
    def _():
        pltpu.sync_copy(zeros_hbm.at[pl.ds(NS * ROWS_N, ROWS_TAIL)],
                        table_sh.at[pl.ds(NS * ROWS_N, ROWS_TAIL)])

    plsc.subcore_barrier()

    @pl.loop(0, SITER)
    def _(i):
        gid = wid + i * NW

        @pl.when(gid < SGROUPS)
        def _():
            for j in range(SK):
                pltpu.sync_copy(
                    dstix_hbm.at[pl.ds(gid * (SK * CH) + j * CH, CH)],
                    idx_v[j])
            cp = pltpu.async_copy(
                attn_hbm.at[pl.ds(gid * (SK * CH), SK * CH)], rows_v, sem)
            cp.wait()
            for j in range(SK):
                pltpu.sync_copy(rows_v.at[pl.ds(j * CH, CH)],
                                table_sh.at[idx_v[j]], add=True)

    plsc.subcore_barrier()
    pltpu.sync_copy(table_sh.at[pl.ds(s * ROWS_N, ROWS_N)],
                    out_hbm.at[c, pl.ds(s * ROWS_N, ROWS_N)])

    @pl.when(s == 0)
    def _():
        pltpu.sync_copy(table_sh.at[pl.ds(NS * ROWS_N, ROWS_TAIL)],
                        out_hbm.at[c, pl.ds(NS * ROWS_N, ROWS_TAIL)])


def _sc_scatter(attn, dst1d, zeros_tab):
    mesh = plsc.VectorSubcoreMesh(core_axis_name="c", subcore_axis_name="s")
    f = pl.kernel(
        _sc_scatter_body,
        out_type=jax.ShapeDtypeStruct((NC, N, D), _F32),
        mesh=mesh,
        scratch_types=(
            [pltpu.VMEM((CH,), jnp.int32) for _ in range(SK)] + [
                pltpu.VMEM((SK * CH, D), _F32),
                pltpu.VMEM_SHARED((N, D), _F32),
                pltpu.SemaphoreType.DMA,
            ]
        ),
    )
    return f(attn, dst1d, zeros_tab)


WK = 5
WGROUPS = NCHUNK // WK
WITER = (WGROUPS + NW - 1) // NW
NP = N * H
NPP = 40960


def _sc_wsum_body(wexp_hbm, dstix_hbm, wout_hbm, *rest):
    idx_v = rest[0:WK]
    wrows_v, wtab_v, sem = rest[WK:]
    c = lax.axis_index("c")
    s = lax.axis_index("s")
    wid = s * NC + c

    zvec = jnp.zeros((16,), _F32)

    @pl.loop(0, NPP // 16)
    def _(i):
        wtab_v[pl.ds(i * 16, 16)] = zvec

    lane = lax.iota(jnp.int32, 16)

    @pl.loop(0, WITER)
    def _(i):
        gid = wid + i * NW

        @pl.when(gid < WGROUPS)
        def _():
            for j in range(WK):
                pltpu.sync_copy(
                    dstix_hbm.at[pl.ds(gid * (WK * CH) + j * CH, CH)],
                    idx_v[j])
            cpw = pltpu.async_copy(
                wexp_hbm.at[pl.ds(gid * (WK * CH * DE), WK * CH * DE)],
                wrows_v, sem)
            cpw.wait()
            for j in range(WK):
                @pl.loop(0, CH)
                def _(e):
                    x = wrows_v[pl.ds(j * (CH * DE) + e * DE, 16)]
                    dstb = plsc.load_gather(
                        idx_v[j], [jnp.full((16,), 0, jnp.int32) + e])
                    plsc.addupdate_scatter(wtab_v, [dstb * H + lane], x)

    pltpu.sync_copy(wtab_v, wout_hbm.at[pl.ds(wid * NPP, NPP)])


def _sc_wsum(wexp1d, dst1d):
    mesh = plsc.VectorSubcoreMesh(core_axis_name="c", subcore_axis_name="s")
    cp = pltpu.CompilerParams()
    if "needs_layout_passes" in pltpu.CompilerParams.__dataclass_fields__:
        cp = dataclasses.replace(cp, needs_layout_passes=False)
    f = pl.kernel(
        _sc_wsum_body,
        compiler_params=cp,
        out_type=jax.ShapeDtypeStruct((NW * NPP,), _F32),
        mesh=mesh,
        scratch_types=(
            [pltpu.VMEM((CH,), jnp.int32) for _ in range(WK)] + [
                pltpu.VMEM((WK * CH * DE,), _F32),
                pltpu.VMEM((NPP,), _F32),
                pltpu.SemaphoreType.DMA,
            ]
        ),
    )
    return f(wexp1d, dst1d)


def _final_body(acc0_ref, acc1_ref, wparts_ref, node_ref, wprojT_ref,
                brep_ref, lt_ref, rsel_ref, out_ref):
    ssum = acc0_ref[...] + acc1_ref[...]
    w2 = jnp.sum(wparts_ref[...], axis=0)
    inv2 = 1.0 / (w2 + 1e-16)
    rep = jnp.dot(brep_ref[...], inv2, preferred_element_type=_F32)
    inv128 = jnp.dot(rep * lt_ref[...], rsel_ref[...],
                     preferred_element_type=_F32)
    out_ref[...] = node_ref[...] + jnp.dot(
        ssum * inv128, wprojT_ref[...], preferred_element_type=_F32)


def _final_pass(acc0, acc1, wparts2d, node, wprojT, brep, lt, rsel):
    full = lambda shape: pl.BlockSpec(shape, lambda i: tuple(0 for _ in shape))
    wrows = (BNF * H) // 128
    return pl.pallas_call(
        _final_body,
        grid=(pl.cdiv(N, BNF),),
        in_specs=[
            pl.BlockSpec((BNF, D), lambda i: (i, 0)),
            pl.BlockSpec((BNF, D), lambda i: (i, 0)),
            pl.BlockSpec((NW, wrows, 128), lambda i: (0, i, 0)),
            pl.BlockSpec((BNF, D), lambda i: (i, 0)),
            full((D, D)),
            full((BNF, wrows)),
            full((BNF, D)),
            full((D, D)),
        ],
        out_specs=pl.BlockSpec((BNF, D), lambda i: (i, 0)),
        out_shape=jax.ShapeDtypeStruct((N, D), _F32),
    )(acc0, acc1, wparts2d, node, wprojT, brep, lt, rsel)


@jax.jit
def kernel(node, edge_index, rbf, rsh, W_src, W_dst, W_rbf, W_alpha,
           alpha_dot, W_sepval, w_dtp2, W_vlin, W_proj):
    src1d = edge_index[0]
    dst1d = edge_index[1]
    eye4 = jnp.eye(H, dtype=_F32)
    e4 = jnp.repeat(eye4, A, axis=1)
    e4pad = jnp.concatenate(
        [eye4, jnp.zeros((H, DE - H), _F32)], axis=1)
    dvec = (alpha_dot[0][:, :, None] * eye4[:, None, :]).reshape(H * A, H)
    zeros_tab = jnp.zeros((N, D), _F32)

    msrc, mdst = _node_pass(node, W_src.T, W_dst.T)

    g1, g2 = _sc_gather(msrc, mdst, src1d, dst1d)
    attn, wexp = _edge_pass(g1, g2, rbf, rsh, W_rbf.T, W_alpha.T, dvec,
                            W_sepval.T, w_dtp2.reshape(DE, 1), W_vlin.T, e4,
                            e4pad)
    acc = _sc_scatter(attn, dst1d, zeros_tab)
    wparts = _sc_wsum(wexp.reshape(E * DE), dst1d)
    wrows = (BNF * H) // 128
    brep = jnp.repeat(jnp.eye(wrows, dtype=_F32), 128 // H, axis=0)
    lt = jnp.tile(jnp.repeat(jnp.eye(128 // H, dtype=_F32), H, axis=1),
                  (BNF // (128 // H), 1))
    rsel = jnp.tile(jnp.repeat(eye4, A, axis=1), (A, 1))
    return _final_pass(acc[0], acc[1], wparts.reshape(NW, NPP // 128, 128),
                       node, W_proj.T, brep, lt, rsel)
